# Initial kernel scaffold; baseline (speedup 1.0000x reference)
import functools

import jax
import jax.numpy as jnp
from jax import lax
from jax.experimental import pallas as pl
from jax.experimental.pallas import tpu as pltpu
from jax.experimental.pallas import tpu_sc as plsc

N = 10000
E = 160000
D = 256
HALF = D // 2
NC = 2
NS = 16
CHUNK = 128
RPT = N // NS

_mesh = plsc.VectorSubcoreMesh(core_axis_name="c", subcore_axis_name="s")


@functools.partial(
    pl.kernel,
    out_type=jax.ShapeDtypeStruct((NC, N, 16), jnp.float32),
    mesh=_mesh,
    scratch_types=[
        pltpu.VMEM((CHUNK,), jnp.int32),
        pltpu.VMEM((CHUNK, 16), jnp.float32),
        pltpu.VMEM_SHARED((N, 16), jnp.float32),
    ],
)
def _sc_bincount(src_hbm, zeros_hbm, ones_hbm, out_hbm, idxv, onesv, acc):
    c = lax.axis_index("c")
    s = lax.axis_index("s")
    row0 = s * RPT
    pltpu.sync_copy(zeros_hbm, acc.at[pl.ds(row0, RPT)])
    pltpu.sync_copy(ones_hbm, onesv)
    plsc.subcore_barrier()

    ehalf = E // NC
    nch = ehalf // CHUNK
    per_tile = (nch + NS - 1) // NS

    @pl.loop(0, per_tile)
    def _(t):
        k = t * NS + s

        @pl.when(k < nch)
        def _():
            base = c * ehalf + k * CHUNK
            pltpu.sync_copy(src_hbm.at[pl.ds(base, CHUNK)], idxv)
            pltpu.sync_copy(onesv, acc.at[idxv], add=True)

    plsc.subcore_barrier()
    pltpu.sync_copy(acc.at[pl.ds(row0, RPT)], out_hbm.at[c, pl.ds(row0, RPT)])


@functools.partial(
    pl.kernel,
    out_type=jax.ShapeDtypeStruct((NC, N, HALF), jnp.float32),
    mesh=_mesh,
    scratch_types=[
        pltpu.VMEM((CHUNK,), jnp.int32),
        pltpu.VMEM((CHUNK,), jnp.int32),
        pltpu.VMEM((CHUNK,), jnp.int32),
        pltpu.VMEM((CHUNK, HALF), jnp.float32),
        pltpu.VMEM_SHARED((N, HALF), jnp.float32),
        pltpu.SemaphoreType.DMA,
    ],
)
def _sc_scatter(g_hbm, src_hbm, dst_hbm, zeros_hbm, out_hbm,
                srcv, dstv, gidx, gbuf, acc, sem):
    c = lax.axis_index("c")
    s = lax.axis_index("s")
    row0 = s * RPT
    pltpu.sync_copy(zeros_hbm, acc.at[pl.ds(row0, RPT)])
    plsc.subcore_barrier()

    nch = E // CHUNK
    per_tile = (nch + NS - 1) // NS

    @pl.loop(0, per_tile)
    def _(t):
        k = t * NS + s

        @pl.when(k < nch)
        def _():
            base = k * CHUNK
            pltpu.sync_copy(src_hbm.at[pl.ds(base, CHUNK)], srcv)
            pltpu.sync_copy(dst_hbm.at[pl.ds(base, CHUNK)], dstv)

            @pl.loop(0, CHUNK // 16)
            def _(j):
                v = srcv[pl.ds(j * 16, 16)]
                gidx[pl.ds(j * 16, 16)] = v * 2 + c

            pltpu.async_copy(g_hbm.at[gidx], gbuf, sem).wait()
            pltpu.sync_copy(gbuf, acc.at[dstv], add=True)

    plsc.subcore_barrier()
    pltpu.sync_copy(acc.at[pl.ds(row0, RPT)], out_hbm.at[c, pl.ds(row0, RPT)])


_ROWS = 1000


def _tc_main_body(counts_ref, x_ref, w_ref, b_ref, root_ref, g_ref, self_ref):
    cnt = counts_ref[0, :, 0:1] + counts_ref[1, :, 0:1]
    degs = cnt + 1.0
    norm = lax.rsqrt(degs)
    h = jnp.dot(x_ref[...], w_ref[...],
                preferred_element_type=jnp.float32) + b_ref[...]
    g_ref[...] = norm * jnp.maximum(h, 0.0)
    self_ref[...] = jnp.maximum(h + root_ref[...], 0.0) / degs


def _tc_main(counts, x, w, b2, root):
    return pl.pallas_call(
        _tc_main_body,
        grid=(N // _ROWS,),
        in_specs=[
            pl.BlockSpec((NC, _ROWS, 16), lambda i: (0, i, 0)),
            pl.BlockSpec((_ROWS, D), lambda i: (i, 0)),
            pl.BlockSpec((D, D), lambda i: (0, 0)),
            pl.BlockSpec((1, D), lambda i: (0, 0)),
            pl.BlockSpec((1, D), lambda i: (0, 0)),
        ],
        out_specs=[
            pl.BlockSpec((_ROWS, D), lambda i: (i, 0)),
            pl.BlockSpec((_ROWS, D), lambda i: (i, 0)),
        ],
        out_shape=[
            jax.ShapeDtypeStruct((N, D), jnp.float32),
            jax.ShapeDtypeStruct((N, D), jnp.float32),
        ],
    )(counts, x, w, b2, root)


def _tc_out_body(counts_ref, acc_ref, self_ref, o_ref):
    cnt = counts_ref[0, :, 0:1] + counts_ref[1, :, 0:1]
    norm = lax.rsqrt(cnt + 1.0)
    acc = jnp.concatenate([acc_ref[0], acc_ref[1]], axis=1)
    o_ref[...] = norm * acc + self_ref[...]


def _tc_out(counts, acc, self_term):
    return pl.pallas_call(
        _tc_out_body,
        grid=(N // _ROWS,),
        in_specs=[
            pl.BlockSpec((NC, _ROWS, 16), lambda i: (0, i, 0)),
            pl.BlockSpec((NC, _ROWS, HALF), lambda i: (0, i, 0)),
            pl.BlockSpec((_ROWS, D), lambda i: (i, 0)),
        ],
        out_specs=pl.BlockSpec((_ROWS, D), lambda i: (i, 0)),
        out_shape=jax.ShapeDtypeStruct((N, D), jnp.float32),
    )(counts, acc, self_term)


def kernel(x, edge_index, W, b, root_emb):
    src = edge_index[0]
    dst = edge_index[1]
    zeros16 = jnp.zeros((RPT, 16), jnp.float32)
    ones16 = jnp.ones((CHUNK, 16), jnp.float32)
    zeros128 = jnp.zeros((RPT, HALF), jnp.float32)

    counts = _sc_bincount(src, zeros16, ones16)
    g, self_term = _tc_main(counts, x, W, b.reshape(1, D), root_emb)
    acc = _sc_scatter(g.reshape(NC * N, HALF), src, dst, zeros128)
    return _tc_out(counts, acc, self_term)


# speedup vs baseline: 10.1265x; 10.1265x over previous
import functools

import jax
import jax.numpy as jnp
from jax import lax
from jax.experimental import pallas as pl
from jax.experimental.pallas import tpu as pltpu
from jax.experimental.pallas import tpu_sc as plsc

N = 10000
E = 160000
D = 256
HALF = D // 2
NC = 2
NS = 16
CHUNK = 128
RPT = 640
NPAD = NS * RPT


@functools.cache
def _sc_kernels():
    mesh = plsc.VectorSubcoreMesh(core_axis_name="c", subcore_axis_name="s")

    @functools.partial(
        pl.kernel,
        out_type=jax.ShapeDtypeStruct((NC, NPAD, 16), jnp.float32),
        mesh=mesh,
        scratch_types=[
            pltpu.VMEM((CHUNK,), jnp.int32),
            pltpu.VMEM((CHUNK, 16), jnp.float32),
            pltpu.VMEM_SHARED((NPAD, 16), jnp.float32),
        ],
    )
    def sc_bincount(src_hbm, zeros_hbm, ones_hbm, out_hbm, idxv, onesv, acc):
        c = lax.axis_index("c")
        s = lax.axis_index("s")
        row0 = s * RPT
        pltpu.sync_copy(zeros_hbm, acc.at[pl.ds(row0, RPT)])
        pltpu.sync_copy(ones_hbm, onesv)
        plsc.subcore_barrier()

        ehalf = E // NC
        nch = ehalf // CHUNK
        per_tile = (nch + NS - 1) // NS

        @pl.loop(0, per_tile)
        def _(t):
            k = t * NS + s

            @pl.when(k < nch)
            def _():
                base = c * ehalf + k * CHUNK
                pltpu.sync_copy(src_hbm.at[pl.ds(base, CHUNK)], idxv)
                pltpu.sync_copy(onesv, acc.at[idxv], add=True)

        plsc.subcore_barrier()
        pltpu.sync_copy(acc.at[pl.ds(row0, RPT)],
                        out_hbm.at[c, pl.ds(row0, RPT)])

    @functools.partial(
        pl.kernel,
        out_type=jax.ShapeDtypeStruct((NC, NPAD, HALF), jnp.float32),
        mesh=mesh,
        scratch_types=[
            pltpu.VMEM((CHUNK,), jnp.int32),
            pltpu.VMEM((CHUNK,), jnp.int32),
            pltpu.VMEM((CHUNK,), jnp.int32),
            pltpu.VMEM((CHUNK, HALF), jnp.float32),
            pltpu.VMEM_SHARED((NPAD, HALF), jnp.float32),
            pltpu.SemaphoreType.DMA,
        ],
    )
    def sc_scatter(g_hbm, src_hbm, dst_hbm, zeros_hbm, out_hbm,
                   srcv, dstv, gidx, gbuf, acc, sem):
        c = lax.axis_index("c")
        s = lax.axis_index("s")
        row0 = s * RPT
        pltpu.sync_copy(zeros_hbm, acc.at[pl.ds(row0, RPT)])
        plsc.subcore_barrier()

        nch = E // CHUNK
        per_tile = (nch + NS - 1) // NS

        @pl.loop(0, per_tile)
        def _(t):
            k = t * NS + s

            @pl.when(k < nch)
            def _():
                base = k * CHUNK
                pltpu.sync_copy(src_hbm.at[pl.ds(base, CHUNK)], srcv)
                pltpu.sync_copy(dst_hbm.at[pl.ds(base, CHUNK)], dstv)

                @pl.loop(0, CHUNK // 16)
                def _(j):
                    v = srcv[pl.ds(j * 16, 16)]
                    gidx[pl.ds(j * 16, 16)] = v * 2 + c

                pltpu.async_copy(g_hbm.at[gidx], gbuf, sem).wait()
                pltpu.sync_copy(gbuf, acc.at[dstv], add=True)

        plsc.subcore_barrier()
        pltpu.sync_copy(acc.at[pl.ds(row0, RPT)],
                        out_hbm.at[c, pl.ds(row0, RPT)])

    return sc_bincount, sc_scatter


_ROWS = 1000


def _tc_main_body(counts_ref, x_ref, w_ref, b_ref, root_ref, g_ref, self_ref):
    cnt = counts_ref[0, :, 0:1] + counts_ref[1, :, 0:1]
    degs = cnt + 1.0
    norm = lax.rsqrt(degs)
    h = jnp.dot(x_ref[...], w_ref[...],
                preferred_element_type=jnp.float32) + b_ref[...]
    g_ref[...] = norm * jnp.maximum(h, 0.0)
    self_ref[...] = jnp.maximum(h + root_ref[...], 0.0) / degs


def _tc_main(counts, x, w, b2, root):
    return pl.pallas_call(
        _tc_main_body,
        grid=(N // _ROWS,),
        in_specs=[
            pl.BlockSpec((NC, _ROWS, 16), lambda i: (0, i, 0)),
            pl.BlockSpec((_ROWS, D), lambda i: (i, 0)),
            pl.BlockSpec((D, D), lambda i: (0, 0)),
            pl.BlockSpec((1, D), lambda i: (0, 0)),
            pl.BlockSpec((1, D), lambda i: (0, 0)),
        ],
        out_specs=[
            pl.BlockSpec((_ROWS, D), lambda i: (i, 0)),
            pl.BlockSpec((_ROWS, D), lambda i: (i, 0)),
        ],
        out_shape=[
            jax.ShapeDtypeStruct((N, D), jnp.float32),
            jax.ShapeDtypeStruct((N, D), jnp.float32),
        ],
    )(counts, x, w, b2, root)


def _tc_out_body(counts_ref, acc_ref, self_ref, o_ref):
    cnt = counts_ref[0, :, 0:1] + counts_ref[1, :, 0:1]
    norm = lax.rsqrt(cnt + 1.0)
    acc = jnp.concatenate([acc_ref[0], acc_ref[1]], axis=1)
    o_ref[...] = norm * acc + self_ref[...]


def _tc_out(counts, acc, self_term):
    return pl.pallas_call(
        _tc_out_body,
        grid=(N // _ROWS,),
        in_specs=[
            pl.BlockSpec((NC, _ROWS, 16), lambda i: (0, i, 0)),
            pl.BlockSpec((NC, _ROWS, HALF), lambda i: (0, i, 0)),
            pl.BlockSpec((_ROWS, D), lambda i: (i, 0)),
        ],
        out_specs=pl.BlockSpec((_ROWS, D), lambda i: (i, 0)),
        out_shape=jax.ShapeDtypeStruct((N, D), jnp.float32),
    )(counts, acc, self_term)


def kernel(x, edge_index, W, b, root_emb):
    sc_bincount, sc_scatter = _sc_kernels()
    src = edge_index[0]
    dst = edge_index[1]
    zeros16 = jnp.zeros((RPT, 16), jnp.float32)
    ones16 = jnp.ones((CHUNK, 16), jnp.float32)
    zeros128 = jnp.zeros((RPT, HALF), jnp.float32)

    counts = sc_bincount(src, zeros16, ones16)
    g, self_term = _tc_main(counts, x, W, b.reshape(1, D), root_emb)
    acc = sc_scatter(g.reshape(NC * N, HALF), src, dst, zeros128)
    return _tc_out(counts, acc, self_term)
